# baseline (speedup 1.0000x reference)
import jax
import jax.numpy as jnp
from jax.experimental import pallas as pl
from jax.experimental.pallas import tpu as pltpu

_T = 8
_G = 4


def _interlace_body(x_ref, cwm_ref, wcw0_ref, wcw1_ref, f1w_ref, f2w_ref,
                    cb_ref, f1b_ref, f2b_ref, wcb_ref, o_ref):
    j = pl.program_id(1)

    @pl.when(j != 0)
    def _copy():
        o_ref[...] = x_ref[...]

    @pl.when(j == 0)
    def _compute():
        data = x_ref[0]
        nf = data.shape[1]
        gc = nf // _G
        pooled = jnp.mean(data, axis=2)

        def conv_t(mm, bias):
            a = mm[:, 0:1]
            b = mm[:, 1:2]
            c = mm[:, 2:3]
            z = jnp.zeros((1, 1), jnp.float32)
            return (b + jnp.concatenate([z, a[:-1]], axis=0)
                    + jnp.concatenate([c[1:], z], axis=0) + bias)

        mm = jnp.dot(pooled, cwm_ref[...], preferred_element_type=jnp.float32)
        oc = conv_t(mm, cb_ref[0, 0])
        h1 = jnp.maximum(
            jnp.dot(f1w_ref[...], oc, preferred_element_type=jnp.float32)
            + f1b_ref[...], 0.0)
        o2 = (jnp.dot(f2w_ref[...], h1, preferred_element_type=jnp.float32)
              + f2b_ref[...])
        offv = 4.0 * (jax.nn.sigmoid(o2) - 0.5)

        wm0 = jnp.dot(pooled, wcw0_ref[...], preferred_element_type=jnp.float32)
        wm1 = jnp.dot(pooled, wcw1_ref[...], preferred_element_type=jnp.float32)
        xw0 = 2.0 * jax.nn.sigmoid(conv_t(wm0, wcb_ref[0, 0]))
        xw1 = 2.0 * jax.nn.sigmoid(conv_t(wm1, wcb_ref[1, 0]))

        for g in range(_G):
            off_g = offv[g % 2, 0]
            if g >= 2:
                off_g = -off_g
            o0f = jnp.floor(off_g)
            o0 = o0f.astype(jnp.int32)
            frac = off_g - o0f
            w0 = 1.0 - frac
            w1 = frac
            xw = xw0 if (g % 2 == 0) else xw1
            for t in range(_T):
                s0 = t + o0
                s1 = s0 + 1
                v0 = jnp.where((s0 >= 0) & (s0 < _T), 1.0, 0.0)
                v1 = jnp.where((s1 >= 0) & (s1 < _T), 1.0, 0.0)
                xwt = xw[t, 0]
                c0 = w0 * xwt * v0
                c1 = w1 * xwt * v1
                s0c = jnp.clip(s0, 0, _T - 1)
                s1c = jnp.clip(s1, 0, _T - 1)
                a0 = x_ref[0, pl.ds(s0c, 1), g * gc:(g + 1) * gc, :]
                a1 = x_ref[0, pl.ds(s1c, 1), g * gc:(g + 1) * gc, :]
                o_ref[0, t, g * gc:(g + 1) * gc, :] = (c0 * a0 + c1 * a1)[0]


def kernel(x, off_conv_w, off_conv_b, off_fc1_w, off_fc1_b, off_fc2_w,
           off_fc2_b, w_conv_w, w_conv_b):
    n, c, h, w = x.shape
    nb = n // _T
    hw = h * w
    nf = c // 4
    xr = x.reshape(nb, _T, c, hw)

    cwm = off_conv_w[0]
    wcw0 = w_conv_w[0]
    wcw1 = w_conv_w[1]
    cb = off_conv_b.reshape(1, 1)
    f1b = off_fc1_b.reshape(_T, 1)
    f2b = off_fc2_b.reshape(2, 1)
    wcb = w_conv_b.reshape(2, 1)

    blk = (1, _T, nf, hw)
    small = lambda shape: pl.BlockSpec(shape, lambda i, j: (0, 0))
    out = pl.pallas_call(
        _interlace_body,
        grid=(nb, c // nf),
        in_specs=[
            pl.BlockSpec(blk, lambda i, j: (i, 0, j, 0)),
            small((nf, 3)), small((nf, 3)), small((nf, 3)),
            small((_T, _T)), small((2, _T)),
            small((1, 1)), small((_T, 1)), small((2, 1)), small((2, 1)),
        ],
        out_specs=pl.BlockSpec(blk, lambda i, j: (i, 0, j, 0)),
        out_shape=jax.ShapeDtypeStruct((nb, _T, c, hw), jnp.float32),
        compiler_params=pltpu.CompilerParams(
            dimension_semantics=("parallel", "arbitrary")),
    )(xr, cwm, wcw0, wcw1, off_fc1_w, off_fc2_w, cb, f1b, f2b, wcb)
    return out.reshape(n, c, h, w)


# speedup vs baseline: 1.0325x; 1.0325x over previous
import jax
import jax.numpy as jnp
from jax.experimental import pallas as pl
from jax.experimental.pallas import tpu as pltpu

_T = 8
_G = 4


def _interlace_body(x_ref, cwm_ref, wcw0_ref, wcw1_ref, f1w_ref, f2w_ref,
                    cb_ref, f1b_ref, f2b_ref, wcb_ref, o_ref):
    nf = x_ref.shape[2] // 4

    o_ref[0, :, nf:, :] = x_ref[0, :, nf:, :]

    def _compute():
        data = x_ref[0, :, :nf, :]
        gc = nf // _G
        pooled = jnp.mean(data, axis=2)

        def conv_t(mm, bias):
            a = mm[:, 0:1]
            b = mm[:, 1:2]
            c = mm[:, 2:3]
            z = jnp.zeros((1, 1), jnp.float32)
            return (b + jnp.concatenate([z, a[:-1]], axis=0)
                    + jnp.concatenate([c[1:], z], axis=0) + bias)

        mm = jnp.dot(pooled, cwm_ref[...], preferred_element_type=jnp.float32)
        oc = conv_t(mm, cb_ref[0, 0])
        h1 = jnp.maximum(
            jnp.dot(f1w_ref[...], oc, preferred_element_type=jnp.float32)
            + f1b_ref[...], 0.0)
        o2 = (jnp.dot(f2w_ref[...], h1, preferred_element_type=jnp.float32)
              + f2b_ref[...])
        offv = 4.0 * (jax.nn.sigmoid(o2) - 0.5)

        wm0 = jnp.dot(pooled, wcw0_ref[...], preferred_element_type=jnp.float32)
        wm1 = jnp.dot(pooled, wcw1_ref[...], preferred_element_type=jnp.float32)
        xw0 = 2.0 * jax.nn.sigmoid(conv_t(wm0, wcb_ref[0, 0]))
        xw1 = 2.0 * jax.nn.sigmoid(conv_t(wm1, wcb_ref[1, 0]))

        for g in range(_G):
            off_g = offv[g % 2, 0]
            if g >= 2:
                off_g = -off_g
            o0f = jnp.floor(off_g)
            o0 = o0f.astype(jnp.int32)
            frac = off_g - o0f
            w0 = 1.0 - frac
            w1 = frac
            xw = xw0 if (g % 2 == 0) else xw1
            for t in range(_T):
                s0 = t + o0
                s1 = s0 + 1
                v0 = jnp.where((s0 >= 0) & (s0 < _T), 1.0, 0.0)
                v1 = jnp.where((s1 >= 0) & (s1 < _T), 1.0, 0.0)
                xwt = xw[t, 0]
                c0 = w0 * xwt * v0
                c1 = w1 * xwt * v1
                s0c = jnp.clip(s0, 0, _T - 1)
                s1c = jnp.clip(s1, 0, _T - 1)
                a0 = x_ref[0, pl.ds(s0c, 1), g * gc:(g + 1) * gc, :]
                a1 = x_ref[0, pl.ds(s1c, 1), g * gc:(g + 1) * gc, :]
                o_ref[0, t, g * gc:(g + 1) * gc, :] = (c0 * a0 + c1 * a1)[0]

    _compute()


def kernel(x, off_conv_w, off_conv_b, off_fc1_w, off_fc1_b, off_fc2_w,
           off_fc2_b, w_conv_w, w_conv_b):
    n, c, h, w = x.shape
    nb = n // _T
    hw = h * w
    nf = c // 4
    xr = x.reshape(nb, _T, c, hw)

    cwm = off_conv_w[0]
    wcw0 = w_conv_w[0]
    wcw1 = w_conv_w[1]
    cb = off_conv_b.reshape(1, 1)
    f1b = off_fc1_b.reshape(_T, 1)
    f2b = off_fc2_b.reshape(2, 1)
    wcb = w_conv_b.reshape(2, 1)

    blk = (1, _T, c, hw)
    small = lambda shape: pl.BlockSpec(shape, lambda i: (0, 0))
    out = pl.pallas_call(
        _interlace_body,
        grid=(nb,),
        in_specs=[
            pl.BlockSpec(blk, lambda i: (i, 0, 0, 0)),
            small((nf, 3)), small((nf, 3)), small((nf, 3)),
            small((_T, _T)), small((2, _T)),
            small((1, 1)), small((_T, 1)), small((2, 1)), small((2, 1)),
        ],
        out_specs=pl.BlockSpec(blk, lambda i: (i, 0, 0, 0)),
        out_shape=jax.ShapeDtypeStruct((nb, _T, c, hw), jnp.float32),
        compiler_params=pltpu.CompilerParams(
            dimension_semantics=("arbitrary",)),
    )(xr, cwm, wcw0, wcw1, off_fc1_w, off_fc2_w, cb, f1b, f2b, wcb)
    return out.reshape(n, c, h, w)
